# Initial kernel scaffold; baseline (speedup 1.0000x reference)
import functools

import jax
import jax.numpy as jnp
from jax import lax
from jax.experimental import pallas as pl
from jax.experimental.pallas import tpu as pltpu
from jax.experimental.pallas import tpu_sc as plsc

N = 10000
E = 320000
D = 128
DO = 64
NP = 10240
ROWS_PER_SUB = NP // 16
CH = 128
NW = 32
PER_W = 79 * CH
EP = NW * PER_W
NCHUNK = PER_W // CH
DW = 16

_mesh = plsc.VectorSubcoreMesh(core_axis_name="c", subcore_axis_name="s")


def _fill_rows(buf, nrows, ncols, value):
    vec = jnp.full((16,), value, jnp.float32)

    @pl.loop(0, nrows)
    def _(r):
        @pl.loop(0, ncols // 16)
        def _(j):
            buf[r, pl.ds(j * 16, 16)] = vec


@functools.partial(
    pl.kernel,
    out_type=jax.ShapeDtypeStruct((2, NP, DW), jnp.float32),
    mesh=_mesh,
    scratch_types=[
        pltpu.VMEM((CH,), jnp.int32),
        pltpu.VMEM((CH, DW), jnp.float32),
        pltpu.VMEM_SHARED((NP, DW), jnp.float32),
    ],
)
def _sc_deg(dst_hbm, out_hbm, dstv, buf, acc):
    c = lax.axis_index("c")
    s = lax.axis_index("s")
    wid = c * 16 + s

    _fill_rows(buf, CH, DW, 0.0)

    @pl.loop(0, ROWS_PER_SUB // CH)
    def _(k):
        pltpu.sync_copy(buf, acc.at[pl.ds(s * ROWS_PER_SUB + k * CH, CH)])

    _fill_rows(buf, CH, DW, 1.0)
    plsc.subcore_barrier()

    @pl.loop(0, NCHUNK)
    def _(i):
        base = wid * PER_W + i * CH
        pltpu.sync_copy(dst_hbm.at[pl.ds(base, CH)], dstv)
        pltpu.sync_copy(buf, acc.at[dstv], add=True)

    plsc.subcore_barrier()
    pltpu.sync_copy(acc.at[pl.ds(s * ROWS_PER_SUB, ROWS_PER_SUB)],
                    out_hbm.at[c, pl.ds(s * ROWS_PER_SUB, ROWS_PER_SUB)])


@functools.partial(
    pl.kernel,
    out_type=jax.ShapeDtypeStruct((2, NP, D), jnp.float32),
    mesh=_mesh,
    scratch_types=[
        pltpu.VMEM((CH,), jnp.int32),
        pltpu.VMEM((CH,), jnp.int32),
        pltpu.VMEM((CH, D), jnp.float32),
        pltpu.VMEM_SHARED((NP, D), jnp.float32),
        pltpu.SemaphoreType.DMA,
    ],
)
def _sc_edges(y_hbm, src_hbm, dst_hbm, out_hbm, srcv, dstv, rows, acc, sem):
    c = lax.axis_index("c")
    s = lax.axis_index("s")
    wid = c * 16 + s

    _fill_rows(rows, CH, D, 0.0)

    @pl.loop(0, ROWS_PER_SUB // CH)
    def _(k):
        pltpu.sync_copy(rows, acc.at[pl.ds(s * ROWS_PER_SUB + k * CH, CH)])

    plsc.subcore_barrier()

    @pl.loop(0, NCHUNK)
    def _(i):
        base = wid * PER_W + i * CH
        pltpu.sync_copy(src_hbm.at[pl.ds(base, CH)], srcv)
        pltpu.sync_copy(dst_hbm.at[pl.ds(base, CH)], dstv)
        pltpu.async_copy(y_hbm.at[srcv], rows, sem).wait()
        pltpu.sync_copy(rows, acc.at[dstv], add=True)

    plsc.subcore_barrier()
    pltpu.sync_copy(acc.at[pl.ds(s * ROWS_PER_SUB, ROWS_PER_SUB)],
                    out_hbm.at[c, pl.ds(s * ROWS_PER_SUB, ROWS_PER_SUB)])


def _row_mask(shape):
    return lax.broadcasted_iota(jnp.int32, shape, 0) < N


def _tc_matmul_body(x_ref, w_ref, o_ref):
    o_ref[...] = jnp.dot(x_ref[...], w_ref[...],
                         preferred_element_type=jnp.float32)


def _tc_matmul(x, w):
    return pl.pallas_call(
        _tc_matmul_body,
        out_shape=jax.ShapeDtypeStruct((x.shape[0], w.shape[1]), jnp.float32),
    )(x, w)


def _tc_prep_body(degp_ref, xw_ref, d_ref, y_ref):
    degp = degp_ref[...]
    deg = degp[0, :, 0:1] + degp[1, :, 0:1] + 1.0
    d = lax.rsqrt(deg)
    d_ref[...] = d
    y = d * xw_ref[...]
    y_ref[...] = jnp.where(_row_mask(y.shape), y, 0.0)


def _tc_prep(deg_parts, xw):
    return pl.pallas_call(
        _tc_prep_body,
        out_shape=(jax.ShapeDtypeStruct((NP, 1), jnp.float32),
                   jax.ShapeDtypeStruct((NP, D), jnp.float32)),
    )(deg_parts, xw)


def _tc_mid_body(sp_ref, y_ref, d_ref, b_ref, w_ref, o_ref):
    sp = sp_ref[...]
    d = d_ref[...]
    h = sp[0] + sp[1] + y_ref[...]
    h = jnp.maximum(d * h + b_ref[...][None, :], 0.0)
    xw = jnp.dot(h, w_ref[...], preferred_element_type=jnp.float32)
    y2 = d * xw
    o_ref[...] = jnp.where(_row_mask(y2.shape), y2, 0.0)


def _tc_mid(s_parts, y, d, b, w):
    return pl.pallas_call(
        _tc_mid_body,
        out_shape=jax.ShapeDtypeStruct((NP, D), jnp.float32),
    )(s_parts, y, d, b, w)


def _tc_final_body(sp_ref, y_ref, d_ref, b_ref, w_ref, bfc_ref, o_ref):
    sp = sp_ref[...]
    h = sp[0] + sp[1] + y_ref[...]
    h = jnp.maximum(d_ref[...] * h + b_ref[...][None, :], 0.0)
    o_ref[...] = (jnp.dot(h, w_ref[...], preferred_element_type=jnp.float32)
                  + bfc_ref[...][None, :])


def _tc_final(s_parts, y, d, b, wfc, bfc):
    return pl.pallas_call(
        _tc_final_body,
        out_shape=jax.ShapeDtypeStruct((NP, DO), jnp.float32),
    )(s_parts, y, d, b, wfc, bfc)


def kernel(x, edge_index, W1, b1, W2, b2, Wfc, bfc):
    x_pad = jnp.pad(x, ((0, NP - N), (0, 0)))
    pad = jnp.full((EP - E,), N, jnp.int32)
    src_pad = jnp.concatenate([edge_index[0], pad])
    dst_pad = jnp.concatenate([edge_index[1], pad])

    deg_parts = _sc_deg(dst_pad)
    xw1 = _tc_matmul(x_pad, W1)
    d, y1 = _tc_prep(deg_parts, xw1)
    s1 = _sc_edges(y1, src_pad, dst_pad)
    y2 = _tc_mid(s1, y1, d, b1, W2)
    s2 = _sc_edges(y2, src_pad, dst_pad)
    out = _tc_final(s2, y2, d, b2, Wfc, bfc)
    return out[:N]


# speedup vs baseline: 10.0127x; 10.0127x over previous
import functools

import jax
import jax.numpy as jnp
from jax import lax
from jax.experimental import pallas as pl
from jax.experimental.pallas import tpu as pltpu
from jax.experimental.pallas import tpu_sc as plsc

N = 10000
E = 320000
D = 128
DO = 64
NP = 10240
ROWS_PER_SUB = NP // 16
CH = 128
NW = 32
PER_W = 79 * CH
EP = NW * PER_W
NCHUNK = PER_W // CH
DW = 128

_mesh = plsc.VectorSubcoreMesh(core_axis_name="c", subcore_axis_name="s")


def _fill_rows(buf, nrows, ncols, value):
    vec = jnp.full((16,), value, jnp.float32)

    @pl.loop(0, nrows)
    def _(r):
        @pl.loop(0, ncols // 16)
        def _(j):
            buf[r, pl.ds(j * 16, 16)] = vec


@functools.partial(
    pl.kernel,
    out_type=jax.ShapeDtypeStruct((2, NP, DW), jnp.float32),
    mesh=_mesh,
    scratch_types=[
        pltpu.VMEM((CH,), jnp.int32),
        pltpu.VMEM((CH, DW), jnp.float32),
        pltpu.VMEM_SHARED((NP, DW), jnp.float32),
    ],
)
def _sc_deg(dst_hbm, out_hbm, dstv, buf, acc):
    c = lax.axis_index("c")
    s = lax.axis_index("s")
    wid = c * 16 + s

    _fill_rows(buf, CH, DW, 0.0)

    @pl.loop(0, ROWS_PER_SUB // CH)
    def _(k):
        pltpu.sync_copy(buf, acc.at[pl.ds(s * ROWS_PER_SUB + k * CH, CH)])

    _fill_rows(buf, CH, DW, 1.0)
    plsc.subcore_barrier()

    @pl.loop(0, NCHUNK)
    def _(i):
        base = wid * PER_W + i * CH
        pltpu.sync_copy(dst_hbm.at[pl.ds(base, CH)], dstv)
        pltpu.sync_copy(buf, acc.at[dstv], add=True)

    plsc.subcore_barrier()
    pltpu.sync_copy(acc.at[pl.ds(s * ROWS_PER_SUB, ROWS_PER_SUB)],
                    out_hbm.at[c, pl.ds(s * ROWS_PER_SUB, ROWS_PER_SUB)])


@functools.partial(
    pl.kernel,
    out_type=jax.ShapeDtypeStruct((2, NP, D), jnp.float32),
    mesh=_mesh,
    scratch_types=[
        pltpu.VMEM((CH,), jnp.int32),
        pltpu.VMEM((CH,), jnp.int32),
        pltpu.VMEM((CH, D), jnp.float32),
        pltpu.VMEM_SHARED((NP, D), jnp.float32),
        pltpu.SemaphoreType.DMA,
    ],
)
def _sc_edges(y_hbm, src_hbm, dst_hbm, out_hbm, srcv, dstv, rows, acc, sem):
    c = lax.axis_index("c")
    s = lax.axis_index("s")
    wid = c * 16 + s

    _fill_rows(rows, CH, D, 0.0)

    @pl.loop(0, ROWS_PER_SUB // CH)
    def _(k):
        pltpu.sync_copy(rows, acc.at[pl.ds(s * ROWS_PER_SUB + k * CH, CH)])

    plsc.subcore_barrier()

    @pl.loop(0, NCHUNK)
    def _(i):
        base = wid * PER_W + i * CH
        pltpu.sync_copy(src_hbm.at[pl.ds(base, CH)], srcv)
        pltpu.sync_copy(dst_hbm.at[pl.ds(base, CH)], dstv)
        pltpu.async_copy(y_hbm.at[srcv], rows, sem).wait()
        pltpu.sync_copy(rows, acc.at[dstv], add=True)

    plsc.subcore_barrier()
    pltpu.sync_copy(acc.at[pl.ds(s * ROWS_PER_SUB, ROWS_PER_SUB)],
                    out_hbm.at[c, pl.ds(s * ROWS_PER_SUB, ROWS_PER_SUB)])


def _row_mask(shape):
    return lax.broadcasted_iota(jnp.int32, shape, 0) < N


def _tc_matmul_body(x_ref, w_ref, o_ref):
    o_ref[...] = jnp.dot(x_ref[...], w_ref[...],
                         preferred_element_type=jnp.float32)


def _tc_matmul(x, w):
    return pl.pallas_call(
        _tc_matmul_body,
        out_shape=jax.ShapeDtypeStruct((x.shape[0], w.shape[1]), jnp.float32),
    )(x, w)


def _tc_prep_body(degp_ref, xw_ref, d_ref, y_ref):
    degp = degp_ref[...]
    deg = degp[0, :, 0:1] + degp[1, :, 0:1] + 1.0
    d = lax.rsqrt(deg)
    d_ref[...] = d
    y = d * xw_ref[...]
    y_ref[...] = jnp.where(_row_mask(y.shape), y, 0.0)


def _tc_prep(deg_parts, xw):
    return pl.pallas_call(
        _tc_prep_body,
        out_shape=(jax.ShapeDtypeStruct((NP, 1), jnp.float32),
                   jax.ShapeDtypeStruct((NP, D), jnp.float32)),
    )(deg_parts, xw)


def _tc_mid_body(sp_ref, y_ref, d_ref, b_ref, w_ref, o_ref):
    sp = sp_ref[...]
    d = d_ref[...]
    h = sp[0] + sp[1] + y_ref[...]
    h = jnp.maximum(d * h + b_ref[...][None, :], 0.0)
    xw = jnp.dot(h, w_ref[...], preferred_element_type=jnp.float32)
    y2 = d * xw
    o_ref[...] = jnp.where(_row_mask(y2.shape), y2, 0.0)


def _tc_mid(s_parts, y, d, b, w):
    return pl.pallas_call(
        _tc_mid_body,
        out_shape=jax.ShapeDtypeStruct((NP, D), jnp.float32),
    )(s_parts, y, d, b, w)


def _tc_final_body(sp_ref, y_ref, d_ref, b_ref, w_ref, bfc_ref, o_ref):
    sp = sp_ref[...]
    h = sp[0] + sp[1] + y_ref[...]
    h = jnp.maximum(d_ref[...] * h + b_ref[...][None, :], 0.0)
    o_ref[...] = (jnp.dot(h, w_ref[...], preferred_element_type=jnp.float32)
                  + bfc_ref[...][None, :])


def _tc_final(s_parts, y, d, b, wfc, bfc):
    return pl.pallas_call(
        _tc_final_body,
        out_shape=jax.ShapeDtypeStruct((NP, DO), jnp.float32),
    )(s_parts, y, d, b, wfc, bfc)


def kernel(x, edge_index, W1, b1, W2, b2, Wfc, bfc):
    x_pad = jnp.pad(x, ((0, NP - N), (0, 0)))
    pad = jnp.full((EP - E,), N, jnp.int32)
    src_pad = jnp.concatenate([edge_index[0], pad])
    dst_pad = jnp.concatenate([edge_index[1], pad])

    deg_parts = _sc_deg(dst_pad)
    xw1 = _tc_matmul(x_pad, W1)
    d, y1 = _tc_prep(deg_parts, xw1)
    s1 = _sc_edges(y1, src_pad, dst_pad)
    y2 = _tc_mid(s1, y1, d, b1, W2)
    s2 = _sc_edges(y2, src_pad, dst_pad)
    out = _tc_final(s2, y2, d, b2, Wfc, bfc)
    return out[:N]
